# baseline (speedup 1.0000x reference)
import jax
import jax.numpy as jnp
from jax import lax
from jax.experimental import pallas as pl
from jax.experimental.pallas import tpu as pltpu
from jax.experimental.pallas import tpu_sc as plsc

BATCH = 16384
DIM = 128
HALF = 64
LANES = 16
NUM_OPS = 1000

_NC = 2
_NS = 16
_NW = _NC * _NS

_ROWS_PER_W = BATCH // _NW
_CHUNK = 128
_NCHUNK = _ROWS_PER_W // _CHUNK


def _pack_body(r_ref, i_ref, o_ref):
    o_ref[:, : HALF] = r_ref[...]
    o_ref[:, HALF:] = i_ref[...]


def _pack(real, imag):
    return pl.pallas_call(
        _pack_body,
        out_shape=jax.ShapeDtypeStruct((NUM_OPS, DIM), jnp.float32),
    )(real, imag)


def _sc_body(emb_hbm, idx_hbm, tab_hbm, out_hbm,
             tab_sh, idx_all, emb_v, tab_v, out_v,
             sem_e, sem_t, sem_o):
    sid = lax.axis_index("s")
    wid = sid * _NC + lax.axis_index("c")
    rbase = wid * _NCHUNK

    @pl.when(sid == 0)
    def _():
        pltpu.sync_copy(tab_hbm, tab_sh)

    pltpu.sync_copy(idx_hbm.at[pl.ds(rbase, _NCHUNK)], idx_all)

    def start_emb(chunk, slot):
        base = (rbase + chunk) * _CHUNK
        pltpu.async_copy(emb_hbm.at[pl.ds(base, _CHUNK)], emb_v.at[slot],
                         sem_e.at[slot])

    def start_gather(chunk, slot):
        pltpu.async_copy(tab_sh.at[idx_all.at[chunk]], tab_v.at[slot],
                         sem_t.at[slot])

    def wait_in(slot):
        pltpu.make_async_copy(emb_hbm.at[pl.ds(0, _CHUNK)], emb_v.at[slot],
                              sem_e.at[slot]).wait()
        pltpu.make_async_copy(tab_sh.at[pl.ds(0, _CHUNK)], tab_v.at[slot],
                              sem_t.at[slot]).wait()

    def wait_out(slot):
        pltpu.make_async_copy(out_v.at[slot], out_hbm.at[pl.ds(0, _CHUNK)],
                              sem_o.at[slot]).wait()

    start_emb(0, 0)
    plsc.subcore_barrier()
    start_gather(0, 0)

    def chunk_body(chunk, carry):
        slot = lax.rem(chunk, 2)
        nslot = 1 - slot

        @pl.when(chunk + 1 < _NCHUNK)
        def _():
            start_emb(chunk + 1, nslot)
            start_gather(chunk + 1, nslot)

        wait_in(slot)

        @pl.when(chunk >= 2)
        def _():
            wait_out(slot)

        @plsc.parallel_loop(0, _CHUNK, 1, unroll=4)
        def row_body(row):
            for c in range(HALF // LANES):
                lo = c * LANES
                hi = HALF + c * LANES
                er = emb_v[slot, row, pl.ds(lo, LANES)]
                ei = emb_v[slot, row, pl.ds(hi, LANES)]
                rb = tab_v[slot, row, pl.ds(lo, LANES)]
                ib = tab_v[slot, row, pl.ds(hi, LANES)]
                out_v[slot, row, pl.ds(lo, LANES)] = er * rb - ei * ib
                out_v[slot, row, pl.ds(hi, LANES)] = er * ib + ei * rb

        base = (rbase + chunk) * _CHUNK
        pltpu.async_copy(out_v.at[slot], out_hbm.at[pl.ds(base, _CHUNK)],
                         sem_o.at[slot])
        return carry

    lax.fori_loop(0, _NCHUNK, chunk_body, 0)
    wait_out(0)
    wait_out(1)


@jax.jit
def _sc_call(embeddings, idx2d, table):
    mesh = plsc.VectorSubcoreMesh(core_axis_name="c", subcore_axis_name="s")
    return pl.kernel(
        _sc_body,
        out_type=jax.ShapeDtypeStruct((BATCH, DIM), jnp.float32),
        mesh=mesh,
        scratch_types=[
            pltpu.VMEM_SHARED((NUM_OPS, DIM), jnp.float32),
            pltpu.VMEM((_NCHUNK, _CHUNK), jnp.int32),
            pltpu.VMEM((2, _CHUNK, DIM), jnp.float32),
            pltpu.VMEM((2, _CHUNK, DIM), jnp.float32),
            pltpu.VMEM((2, _CHUNK, DIM), jnp.float32),
            pltpu.SemaphoreType.DMA((2,)),
            pltpu.SemaphoreType.DMA((2,)),
            pltpu.SemaphoreType.DMA((2,)),
        ],
    )(embeddings, idx2d, table)


def kernel(embeddings, operator_idxs, real, imag):
    idx2d = operator_idxs.astype(jnp.int32).reshape(BATCH // _CHUNK, _CHUNK)
    table = _pack(real, imag)
    return _sc_call(embeddings, idx2d, table)


# speedup vs baseline: 1.0951x; 1.0951x over previous
import jax
import jax.numpy as jnp
from jax import lax
from jax.experimental import pallas as pl
from jax.experimental.pallas import tpu as pltpu
from jax.experimental.pallas import tpu_sc as plsc

BATCH = 16384
DIM = 128
HALF = 64
LANES = 16
NUM_OPS = 1000

_NC = 2
_NS = 16
_NW = _NC * _NS

_ROWS_PER_W = BATCH // _NW
_CHUNK = 128
_NCHUNK = _ROWS_PER_W // _CHUNK


def _sc_body(emb_hbm, idx_hbm, tab_hbm, out_hbm,
             tab_sh, idx_all, emb_v, tab_v, out_v,
             sem_e, sem_t, sem_o):
    sid = lax.axis_index("s")
    wid = sid * _NC + lax.axis_index("c")
    rbase = wid * _NCHUNK

    @pl.when(sid == 0)
    def _():
        pltpu.sync_copy(tab_hbm, tab_sh)

    pltpu.sync_copy(idx_hbm.at[pl.ds(rbase, _NCHUNK)], idx_all)

    def start_emb(chunk, slot):
        base = (rbase + chunk) * _CHUNK
        pltpu.async_copy(emb_hbm.at[pl.ds(base, _CHUNK)], emb_v.at[slot],
                         sem_e.at[slot])

    def start_gather(chunk, slot):
        pltpu.async_copy(tab_sh.at[idx_all.at[chunk]], tab_v.at[slot],
                         sem_t.at[slot])

    def wait_in(slot):
        pltpu.make_async_copy(emb_hbm.at[pl.ds(0, _CHUNK)], emb_v.at[slot],
                              sem_e.at[slot]).wait()
        pltpu.make_async_copy(tab_sh.at[pl.ds(0, _CHUNK)], tab_v.at[slot],
                              sem_t.at[slot]).wait()

    def wait_out(slot):
        pltpu.make_async_copy(out_v.at[slot], out_hbm.at[pl.ds(0, _CHUNK)],
                              sem_o.at[slot]).wait()

    start_emb(0, 0)
    plsc.subcore_barrier()
    start_gather(0, 0)

    def chunk_body(chunk, carry):
        slot = lax.rem(chunk, 2)
        nslot = 1 - slot

        @pl.when(chunk + 1 < _NCHUNK)
        def _():
            start_emb(chunk + 1, nslot)
            start_gather(chunk + 1, nslot)

        wait_in(slot)

        @pl.when(chunk >= 2)
        def _():
            wait_out(slot)

        @plsc.parallel_loop(0, _CHUNK, 1, unroll=4)
        def row_body(row):
            for c in range(HALF // LANES):
                lo = c * LANES
                hi = HALF + c * LANES
                er = emb_v[slot, row, pl.ds(lo, LANES)]
                ei = emb_v[slot, row, pl.ds(hi, LANES)]
                rb = tab_v[slot, row, pl.ds(lo, LANES)]
                ib = tab_v[slot, row, pl.ds(hi, LANES)]
                out_v[slot, row, pl.ds(lo, LANES)] = er * rb - ei * ib
                out_v[slot, row, pl.ds(hi, LANES)] = er * ib + ei * rb

        base = (rbase + chunk) * _CHUNK
        pltpu.async_copy(out_v.at[slot], out_hbm.at[pl.ds(base, _CHUNK)],
                         sem_o.at[slot])
        return carry

    lax.fori_loop(0, _NCHUNK, chunk_body, 0)
    wait_out(0)
    wait_out(1)


@jax.jit
def _sc_call(embeddings, idx2d, table):
    mesh = plsc.VectorSubcoreMesh(core_axis_name="c", subcore_axis_name="s")
    return pl.kernel(
        _sc_body,
        out_type=jax.ShapeDtypeStruct((BATCH, DIM), jnp.float32),
        mesh=mesh,
        scratch_types=[
            pltpu.VMEM_SHARED((NUM_OPS, DIM), jnp.float32),
            pltpu.VMEM((_NCHUNK, _CHUNK), jnp.int32),
            pltpu.VMEM((2, _CHUNK, DIM), jnp.float32),
            pltpu.VMEM((2, _CHUNK, DIM), jnp.float32),
            pltpu.VMEM((2, _CHUNK, DIM), jnp.float32),
            pltpu.SemaphoreType.DMA((2,)),
            pltpu.SemaphoreType.DMA((2,)),
            pltpu.SemaphoreType.DMA((2,)),
        ],
    )(embeddings, idx2d, table)


def kernel(embeddings, operator_idxs, real, imag):
    idx2d = operator_idxs.astype(jnp.int32).reshape(BATCH // _CHUNK, _CHUNK)
    table = jnp.concatenate([real, imag], axis=-1)
    return _sc_call(embeddings, idx2d, table)
